# baseline (speedup 1.0000x reference)
import functools

import jax
import jax.numpy as jnp
from jax.experimental import pallas as pl
from jax.experimental.pallas import tpu as pltpu


def _mean_mid_kernel(x_ref, o_ref, *, inv_r):
    o_ref[...] = jnp.sum(x_ref[...], axis=1) * inv_r


def kernel(x):
    N, C, H, W = x.shape
    R = H * W
    x3 = jnp.transpose(x, (0, 2, 3, 1)).reshape(N, R, C)

    TN = 8
    grid = (N // TN,)

    out = pl.pallas_call(
        functools.partial(_mean_mid_kernel, inv_r=1.0 / R),
        out_shape=jax.ShapeDtypeStruct((N, C), x.dtype),
        grid=grid,
        in_specs=[pl.BlockSpec((TN, R, C), lambda i: (i, 0, 0))],
        out_specs=pl.BlockSpec((TN, C), lambda i: (i, 0)),
        compiler_params=pltpu.CompilerParams(
            dimension_semantics=("parallel",),
            vmem_limit_bytes=64 * 1024 * 1024,
        ),
        cost_estimate=pl.CostEstimate(
            flops=N * R * C,
            transcendentals=0,
            bytes_accessed=N * R * C * 4 + N * C * 4,
        ),
    )(x3)
    return out


# speedup vs baseline: 1.1194x; 1.1194x over previous
import functools

import jax
import jax.numpy as jnp
from jax.experimental import pallas as pl
from jax.experimental.pallas import tpu as pltpu


def _mean_mid_kernel(x_ref, o_ref, *, inv_r):
    o_ref[...] = jnp.sum(x_ref[...], axis=1) * inv_r


def kernel(x):
    N, C, H, W = x.shape
    R = H * W
    x3 = jnp.transpose(x, (0, 2, 3, 1)).reshape(N, R, C)

    TN = 16
    grid = (N // TN,)

    out = pl.pallas_call(
        functools.partial(_mean_mid_kernel, inv_r=1.0 / R),
        out_shape=jax.ShapeDtypeStruct((N, C), x.dtype),
        grid=grid,
        in_specs=[pl.BlockSpec((TN, R, C), lambda i: (i, 0, 0))],
        out_specs=pl.BlockSpec((TN, C), lambda i: (i, 0)),
        compiler_params=pltpu.CompilerParams(
            dimension_semantics=("parallel",),
            vmem_limit_bytes=64 * 1024 * 1024,
        ),
        cost_estimate=pl.CostEstimate(
            flops=N * R * C,
            transcendentals=0,
            bytes_accessed=N * R * C * 4 + N * C * 4,
        ),
    )(x3)
    return out
